# baseline (speedup 1.0000x reference)
import functools

import jax
import jax.numpy as jnp
from jax import lax
from jax.experimental import pallas as pl
from jax.experimental.pallas import tpu as pltpu
from jax.experimental.pallas import tpu_sc as plsc

N = 10000
NPAD = 10240
E = 320000
D_IN = 128
H = 8
DH = 32
DO = 64

KG = 40
NW = 32
EW = E // NW
NG = EW // KG
NGC = 25


def _expand_att(a):
    hh, d = a.shape
    return (a[:, :, None] * jnp.eye(hh, dtype=a.dtype)[:, None, :]).reshape(hh * d, hh)


def _rep_mat(heads, d):
    return jnp.repeat(jnp.eye(heads, dtype=jnp.float32), d, axis=1)


def _wall_mat(heads, fh, nb):
    cols = jnp.arange(nb * 128)
    return (cols[None, :] // fh == jnp.arange(16)[:, None]).astype(jnp.float32)


def _dense1_body(x_ref, w_ref, alm_ref, arm_ref, tb_ref, elt_ref, ert_ref):
    h = jnp.dot(x_ref[...], w_ref[...], preferred_element_type=jnp.float32)
    el = jnp.dot(h, alm_ref[...], preferred_element_type=jnp.float32)
    er = jnp.dot(h, arm_ref[...], preferred_element_type=jnp.float32)
    blk = h.shape[0]
    tb_ref[...] = h.reshape(blk, -1, 128).transpose(1, 0, 2)
    z = jnp.zeros((blk, 120), jnp.float32)
    elt_ref[...] = jnp.concatenate([el, z], axis=1)
    ert_ref[...] = jnp.concatenate([er, z], axis=1)


def _dense1(x, W1, al1, ar1):
    xp = jnp.zeros((NPAD, D_IN), jnp.float32).at[:N].set(x)
    alm = _expand_att(al1)
    arm = _expand_att(ar1)
    nb = (H * DH) // 128
    blk = 1280
    grid = NPAD // blk
    return pl.pallas_call(
        _dense1_body,
        grid=(grid,),
        in_specs=[
            pl.BlockSpec((blk, D_IN), lambda i: (i, 0)),
            pl.BlockSpec((D_IN, H * DH), lambda i: (0, 0)),
            pl.BlockSpec((H * DH, H), lambda i: (0, 0)),
            pl.BlockSpec((H * DH, H), lambda i: (0, 0)),
        ],
        out_specs=[
            pl.BlockSpec((nb, blk, 128), lambda i: (0, i, 0)),
            pl.BlockSpec((blk, 128), lambda i: (i, 0)),
            pl.BlockSpec((blk, 128), lambda i: (i, 0)),
        ],
        out_shape=[
            jax.ShapeDtypeStruct((nb, NPAD, 128), jnp.float32),
            jax.ShapeDtypeStruct((NPAD, 128), jnp.float32),
            jax.ShapeDtypeStruct((NPAD, 128), jnp.float32),
        ],
    )(xp, W1, alm, arm)


def _sc_logits(elt, ert, src3, dst3):
    mesh = plsc.VectorSubcoreMesh(core_axis_name="c", subcore_axis_name="s")

    @functools.partial(
        pl.kernel,
        mesh=mesh,
        out_type=jax.ShapeDtypeStruct((E, 16), jnp.float32),
        scratch_types=[
            pltpu.VMEM((NG, KG), jnp.int32),
            pltpu.VMEM((NG, KG), jnp.int32),
            pltpu.VMEM((2, KG, 128), jnp.float32),
            pltpu.VMEM((2, KG, 128), jnp.float32),
            pltpu.VMEM((2, KG, 16), jnp.float32),
            pltpu.SemaphoreType.DMA,
            pltpu.SemaphoreType.DMA,
            pltpu.SemaphoreType.DMA,
        ],
    )
    def k(elt_h, ert_h, src_h, dst_h, out_h, sidx, didx, ra, rb, sb, gs0, gs1,
          ws):
        wid = lax.axis_index("s") * 2 + lax.axis_index("c")
        gsem = [gs0, gs1]
        pltpu.sync_copy(src_h.at[wid], sidx)
        pltpu.sync_copy(dst_h.at[wid], didx)

        def fire(g, p):
            pltpu.async_copy(elt_h.at[sidx.at[g]], ra.at[p], gsem[p])
            pltpu.async_copy(ert_h.at[didx.at[g]], rb.at[p], gsem[p])

        def process(g, p):
            pltpu.make_async_copy(elt_h.at[sidx.at[g]], ra.at[p], gsem[p]).wait()
            pltpu.make_async_copy(ert_h.at[didx.at[g]], rb.at[p], gsem[p]).wait()

            @pl.when(g >= 2)
            def _():
                pltpu.make_async_copy(
                    sb.at[p], out_h.at[pl.ds(wid * EW, KG)], ws).wait()

            def ebody(e, _):
                v = ra[p, e, pl.ds(0, 16)] + rb[p, e, pl.ds(0, 16)]
                v = jnp.maximum(v, 0.2 * v)
                sb[p, e, pl.ds(0, 16)] = jnp.exp(v)
                return 0

            lax.fori_loop(0, KG, ebody, 0)
            base = wid * EW + g * KG
            pltpu.async_copy(sb.at[p], out_h.at[pl.ds(base, KG)], ws)

        fire(0, 0)

        def pair(i, _):
            for p in range(2):
                g = 2 * i + p

                @pl.when(g + 1 < NG)
                def _(g=g, p=p):
                    fire(g + 1, p ^ 1)

                @pl.when(g < NG)
                def _(g=g, p=p):
                    process(g, p)
            return 0

        lax.fori_loop(0, (NG + 1) // 2, pair, 0)
        for p in range(2):
            pltpu.make_async_copy(
                sb.at[p], out_h.at[pl.ds(wid * EW, KG)], ws).wait()

    return k(elt, ert, src3, dst3)


def _expand_body(w_ref, m_ref, wall_ref):
    wm = jnp.dot(w_ref[...], m_ref[...], preferred_element_type=jnp.float32)
    blk = wm.shape[0]
    wall_ref[...] = wm.reshape(blk, -1, 128).transpose(1, 0, 2)


def _expand(w16, fh, nb):
    m = _wall_mat(H, fh, nb)
    blk = 2000
    grid = E // blk
    return pl.pallas_call(
        _expand_body,
        grid=(grid,),
        in_specs=[
            pl.BlockSpec((blk, 16), lambda i: (i, 0)),
            pl.BlockSpec((16, nb * 128), lambda i: (0, 0)),
        ],
        out_specs=pl.BlockSpec((nb, blk, 128), lambda i: (0, i, 0)),
        out_shape=jax.ShapeDtypeStruct((nb, E, 128), jnp.float32),
    )(w16, m)


def _sc_agg(tb, wall, w16, src, dst, nb):
    mesh = plsc.VectorSubcoreMesh(core_axis_name="c", subcore_axis_name="s")
    rows_per = NPAD // 16
    zrows = jnp.zeros((NPAD, 128), jnp.float32)

    @functools.partial(
        pl.kernel,
        mesh=mesh,
        out_type=jax.ShapeDtypeStruct((nb + 1, 2 * NPAD, 128), jnp.float32),
        scratch_types=[
            pltpu.VMEM_SHARED((NPAD, 128), jnp.float32),
            pltpu.VMEM((NGC, KG), jnp.int32),
            pltpu.VMEM((NGC, KG), jnp.int32),
            pltpu.VMEM((2, KG, 128), jnp.float32),
            pltpu.VMEM((2, KG, 128), jnp.float32),
            pltpu.VMEM((2, KG, 16), jnp.float32),
            pltpu.SemaphoreType.DMA,
            pltpu.SemaphoreType.DMA,
            pltpu.SemaphoreType.DMA,
        ],
    )
    def k(tb_h, wall_h, w16_h, src_h, dst_h, z_h, out_h, acc, sidx, didx,
          rows, wbuf, w16c, gs0, gs1, ss):
        cid = lax.axis_index("c")
        sid = lax.axis_index("s")
        wid = sid * 2 + cid
        myrow = sid * rows_per
        gsem = [gs0, gs1]
        for b in range(nb + 1):
            pltpu.sync_copy(z_h.at[pl.ds(myrow, rows_per)],
                            acc.at[pl.ds(myrow, rows_per)])
            plsc.subcore_barrier()

            gath = b < nb
            if not gath:
                def zbody(e, _):
                    for p in range(2):
                        for q in range(1, 8):
                            rows[p, e, pl.ds(q * 16, 16)] = jnp.zeros(
                                (16,), jnp.float32)
                    return 0

                lax.fori_loop(0, KG, zbody, 0)

            def fire(c, j, p, b=b, gath=gath):
                base = wid * EW + (c * NGC + j) * KG
                if gath:
                    pltpu.async_copy(tb_h.at[b].at[sidx.at[j]], rows.at[p],
                                     gsem[p])
                    pltpu.async_copy(wall_h.at[b].at[pl.ds(base, KG)],
                                     wbuf.at[p], gsem[p])
                else:
                    pltpu.async_copy(w16_h.at[pl.ds(base, KG)],
                                     w16c.at[p], gsem[p])

            def process(c, j, p, b=b, gath=gath):
                base = wid * EW + (c * NGC + j) * KG
                if gath:
                    pltpu.make_async_copy(
                        tb_h.at[b].at[sidx.at[j]], rows.at[p], gsem[p]).wait()
                    pltpu.make_async_copy(
                        wall_h.at[b].at[pl.ds(base, KG)], wbuf.at[p],
                        gsem[p]).wait()

                    def ebody(e, _):
                        for q in range(8):
                            sl = pl.ds(q * 16, 16)
                            rows[p, e, sl] = rows[p, e, sl] * wbuf[p, e, sl]
                        return 0

                    lax.fori_loop(0, KG, ebody, 0)
                else:
                    pltpu.make_async_copy(
                        w16_h.at[pl.ds(base, KG)], w16c.at[p], gsem[p]).wait()

                    def ebody(e, _):
                        rows[p, e, pl.ds(0, 16)] = w16c[p, e, pl.ds(0, 16)]
                        return 0

                    lax.fori_loop(0, KG, ebody, 0)
                pltpu.async_copy(rows.at[p], acc.at[didx.at[j]], ss,
                                 add=True)

            def chunk(c, _):
                pltpu.sync_copy(src_h.at[wid].at[c], sidx)
                pltpu.sync_copy(dst_h.at[wid].at[c], didx)
                fire(c, 0, 0)

                def pair(i, _):
                    for p in range(2):
                        j = 2 * i + p

                        @pl.when(j + 1 < NGC)
                        def _(j=j, p=p):
                            @pl.when(j >= 1)
                            def _():
                                pltpu.make_async_copy(
                                    rows.at[p ^ 1], acc.at[didx.at[0]],
                                    ss).wait()

                            fire(c, j + 1, p ^ 1)

                        @pl.when(j < NGC)
                        def _(j=j, p=p):
                            process(c, j, p)
                    return 0

                lax.fori_loop(0, (NGC + 1) // 2, pair, 0)
                for p in range(2):
                    pltpu.make_async_copy(
                        rows.at[p], acc.at[didx.at[0]], ss).wait()
                return 0

            lax.fori_loop(0, NG // NGC, chunk, 0)

            plsc.subcore_barrier()
            pltpu.sync_copy(
                acc.at[pl.ds(myrow, rows_per)],
                out_h.at[b].at[pl.ds(cid * NPAD + myrow, rows_per)])

    out = k(tb, wall, w16, src, dst, zrows)
    return out.reshape(nb + 1, 2, NPAD, 128)


def _dense2_body(a00, a01, a10, a11, d0, d1, rep_ref, b_ref, w2_ref, alm_ref,
                 arm_ref, tb_ref, elt_ref, ert_ref):
    numer = jnp.concatenate(
        [a00[...] + a01[...], a10[...] + a11[...]], axis=1)
    den8 = (d0[...] + d1[...])[:, :H]
    rep = jnp.dot(den8, rep_ref[...], preferred_element_type=jnp.float32)
    rep = jnp.where(rep == 0.0, 1.0, rep)
    o1 = numer / rep + b_ref[...]
    o1 = jnp.where(o1 > 0, o1, (jnp.exp(o1) - 1.0))
    h2 = jnp.dot(o1, w2_ref[...], preferred_element_type=jnp.float32)
    el = jnp.dot(h2, alm_ref[...], preferred_element_type=jnp.float32)
    er = jnp.dot(h2, arm_ref[...], preferred_element_type=jnp.float32)
    blk = h2.shape[0]
    tb_ref[...] = h2.reshape(blk, -1, 128).transpose(1, 0, 2)
    z = jnp.zeros((blk, 120), jnp.float32)
    elt_ref[...] = jnp.concatenate([el, z], axis=1)
    ert_ref[...] = jnp.concatenate([er, z], axis=1)


def _dense2(agg1, b1, W2, al2, ar2):
    alm = _expand_att(al2)
    arm = _expand_att(ar2)
    rep = _rep_mat(H, DH)
    nb2 = (H * DO) // 128
    blk = 1280
    grid = NPAD // blk
    row = pl.BlockSpec((blk, 128), lambda i: (i, 0))
    return pl.pallas_call(
        _dense2_body,
        grid=(grid,),
        in_specs=[
            row, row, row, row, row, row,
            pl.BlockSpec((H, H * DH), lambda i: (0, 0)),
            pl.BlockSpec((1, H * DH), lambda i: (0, 0)),
            pl.BlockSpec((H * DH, H * DO), lambda i: (0, 0)),
            pl.BlockSpec((H * DO, H), lambda i: (0, 0)),
            pl.BlockSpec((H * DO, H), lambda i: (0, 0)),
        ],
        out_specs=[
            pl.BlockSpec((nb2, blk, 128), lambda i: (0, i, 0)),
            pl.BlockSpec((blk, 128), lambda i: (i, 0)),
            pl.BlockSpec((blk, 128), lambda i: (i, 0)),
        ],
        out_shape=[
            jax.ShapeDtypeStruct((nb2, NPAD, 128), jnp.float32),
            jax.ShapeDtypeStruct((NPAD, 128), jnp.float32),
            jax.ShapeDtypeStruct((NPAD, 128), jnp.float32),
        ],
    )(agg1[0, 0], agg1[0, 1], agg1[1, 0], agg1[1, 1], agg1[2, 0], agg1[2, 1],
      rep, b1.reshape(1, -1), W2, alm, arm)


def _head_body(a00, a01, a10, a11, a20, a21, a30, a31, d0, d1, rep_ref, b_ref,
               w1_ref, b1_ref, w2_ref, b2_ref, w3_ref, b3_ref, out_ref):
    numer = jnp.concatenate(
        [a00[...] + a01[...], a10[...] + a11[...], a20[...] + a21[...],
         a30[...] + a31[...]], axis=1)
    den8 = (d0[...] + d1[...])[:, :H]
    rep = jnp.dot(den8, rep_ref[...], preferred_element_type=jnp.float32)
    rep = jnp.where(rep == 0.0, 1.0, rep)
    o2 = numer / rep + b_ref[...]
    o2 = jnp.where(o2 > 0, o2, (jnp.exp(o2) - 1.0))
    hh = jnp.dot(o2, w1_ref[...], preferred_element_type=jnp.float32) + b1_ref[...]
    hh = jnp.where(hh > 0, hh, 0.01 * hh)
    hh = jnp.dot(hh, w2_ref[...], preferred_element_type=jnp.float32) + b2_ref[...]
    hh = jnp.where(hh > 0, hh, 0.01 * hh)
    out_ref[...] = jnp.dot(hh, w3_ref[...], preferred_element_type=jnp.float32) + b3_ref[...]


def _head(agg2, b2, d1w, d1b, d2w, d2b, d3w, d3b):
    rep = _rep_mat(H, DO)
    blk = 400
    grid = N // blk
    row = pl.BlockSpec((blk, 128), lambda i: (i, 0))
    return pl.pallas_call(
        _head_body,
        grid=(grid,),
        in_specs=[
            row, row, row, row, row, row, row, row, row, row,
            pl.BlockSpec((H, H * DO), lambda i: (0, 0)),
            pl.BlockSpec((1, H * DO), lambda i: (0, 0)),
            pl.BlockSpec((H * DO, DO), lambda i: (0, 0)),
            pl.BlockSpec((1, DO), lambda i: (0, 0)),
            pl.BlockSpec((DO, DO // 2), lambda i: (0, 0)),
            pl.BlockSpec((1, DO // 2), lambda i: (0, 0)),
            pl.BlockSpec((DO // 2, 1), lambda i: (0, 0)),
            pl.BlockSpec((1, 1), lambda i: (0, 0)),
        ],
        out_specs=pl.BlockSpec((blk, 1), lambda i: (i, 0)),
        out_shape=jax.ShapeDtypeStruct((N, 1), jnp.float32),
    )(agg2[0, 0][:N], agg2[0, 1][:N], agg2[1, 0][:N], agg2[1, 1][:N],
      agg2[2, 0][:N], agg2[2, 1][:N], agg2[3, 0][:N], agg2[3, 1][:N],
      agg2[4, 0][:N], agg2[4, 1][:N], rep, b2.reshape(1, -1), d1w,
      d1b.reshape(1, -1), d2w, d2b.reshape(1, -1), d3w, d3b.reshape(1, -1))


def kernel(x, edge_index, W1, al1, ar1, b1, s1w1, s1b1, s1w2, W2, al2, ar2, b2,
           s2w1, s2b1, s2w2, d1w, d1b, d2w, d2b, d3w, d3b):
    src3 = edge_index[0].reshape(NW, NG, KG)
    dst3 = edge_index[1].reshape(NW, NG, KG)
    src4 = edge_index[0].reshape(NW, NG // NGC, NGC, KG)
    dst4 = edge_index[1].reshape(NW, NG // NGC, NGC, KG)
    nb1 = (H * DH) // 128
    nb2 = (H * DO) // 128

    tb1, elt1, ert1 = _dense1(x, W1, al1, ar1)
    w16_1 = _sc_logits(elt1, ert1, src3, dst3)
    wall1 = _expand(w16_1, DH, nb1)
    agg1 = _sc_agg(tb1, wall1, w16_1, src4, dst4, nb1)

    tb2, elt2, ert2 = _dense2(agg1, b1, W2, al2, ar2)
    w16_2 = _sc_logits(elt2, ert2, src3, dst3)
    wall2 = _expand(w16_2, DO, nb2)
    agg2 = _sc_agg(tb2, wall2, w16_2, src4, dst4, nb2)

    return _head(agg2, b2, d1w, d1b, d2w, d2b, d3w, d3b)


# speedup vs baseline: 1.0284x; 1.0284x over previous
import functools

import jax
import jax.numpy as jnp
from jax import lax
from jax.experimental import pallas as pl
from jax.experimental.pallas import tpu as pltpu
from jax.experimental.pallas import tpu_sc as plsc

N = 10000
NPAD = 10240
E = 320000
D_IN = 128
H = 8
DH = 32
DO = 64

KG = 40
NW = 32
EW = E // NW
NG = EW // KG
KGA = 40
NGA = EW // KGA
NGC = 50


def _expand_att(a):
    hh, d = a.shape
    return (a[:, :, None] * jnp.eye(hh, dtype=a.dtype)[:, None, :]).reshape(hh * d, hh)


def _rep_mat(heads, d):
    return jnp.repeat(jnp.eye(heads, dtype=jnp.float32), d, axis=1)


def _wall_mat(heads, fh, nb):
    cols = jnp.arange(nb * 128)
    return (cols[None, :] // fh == jnp.arange(16)[:, None]).astype(jnp.float32)


def _dense1_body(x_ref, w_ref, alm_ref, arm_ref, tb_ref, elt_ref, ert_ref):
    h = jnp.dot(x_ref[...], w_ref[...], preferred_element_type=jnp.float32)
    el = jnp.dot(h, alm_ref[...], preferred_element_type=jnp.float32)
    er = jnp.dot(h, arm_ref[...], preferred_element_type=jnp.float32)
    blk = h.shape[0]
    tb_ref[...] = h.reshape(blk, -1, 128).transpose(1, 0, 2)
    z = jnp.zeros((blk, 120), jnp.float32)
    elt_ref[...] = jnp.concatenate([el, z], axis=1)
    ert_ref[...] = jnp.concatenate([er, z], axis=1)


def _dense1(x, W1, al1, ar1):
    xp = jnp.zeros((NPAD, D_IN), jnp.float32).at[:N].set(x)
    alm = _expand_att(al1)
    arm = _expand_att(ar1)
    nb = (H * DH) // 128
    blk = 1280
    grid = NPAD // blk
    return pl.pallas_call(
        _dense1_body,
        grid=(grid,),
        in_specs=[
            pl.BlockSpec((blk, D_IN), lambda i: (i, 0)),
            pl.BlockSpec((D_IN, H * DH), lambda i: (0, 0)),
            pl.BlockSpec((H * DH, H), lambda i: (0, 0)),
            pl.BlockSpec((H * DH, H), lambda i: (0, 0)),
        ],
        out_specs=[
            pl.BlockSpec((nb, blk, 128), lambda i: (0, i, 0)),
            pl.BlockSpec((blk, 128), lambda i: (i, 0)),
            pl.BlockSpec((blk, 128), lambda i: (i, 0)),
        ],
        out_shape=[
            jax.ShapeDtypeStruct((nb, NPAD, 128), jnp.float32),
            jax.ShapeDtypeStruct((NPAD, 128), jnp.float32),
            jax.ShapeDtypeStruct((NPAD, 128), jnp.float32),
        ],
    )(xp, W1, alm, arm)


def _sc_logits(elt, ert, src3, dst3):
    mesh = plsc.VectorSubcoreMesh(core_axis_name="c", subcore_axis_name="s")

    @functools.partial(
        pl.kernel,
        mesh=mesh,
        out_type=jax.ShapeDtypeStruct((E, 16), jnp.float32),
        scratch_types=[
            pltpu.VMEM((NG, KG), jnp.int32),
            pltpu.VMEM((NG, KG), jnp.int32),
            pltpu.VMEM((2, KG, 128), jnp.float32),
            pltpu.VMEM((2, KG, 128), jnp.float32),
            pltpu.VMEM((2, KG, 16), jnp.float32),
            pltpu.SemaphoreType.DMA,
            pltpu.SemaphoreType.DMA,
            pltpu.SemaphoreType.DMA,
        ],
    )
    def k(elt_h, ert_h, src_h, dst_h, out_h, sidx, didx, ra, rb, sb, gs0, gs1,
          ws):
        wid = lax.axis_index("s") * 2 + lax.axis_index("c")
        gsem = [gs0, gs1]
        pltpu.sync_copy(src_h.at[wid], sidx)
        pltpu.sync_copy(dst_h.at[wid], didx)

        def fire(g, p):
            pltpu.async_copy(elt_h.at[sidx.at[g]], ra.at[p], gsem[p])
            pltpu.async_copy(ert_h.at[didx.at[g]], rb.at[p], gsem[p])

        def process(g, p):
            pltpu.make_async_copy(elt_h.at[sidx.at[g]], ra.at[p], gsem[p]).wait()
            pltpu.make_async_copy(ert_h.at[didx.at[g]], rb.at[p], gsem[p]).wait()

            @pl.when(g >= 2)
            def _():
                pltpu.make_async_copy(
                    sb.at[p], out_h.at[pl.ds(wid * EW, KG)], ws).wait()

            def ebody(e, _):
                v = ra[p, e, pl.ds(0, 16)] + rb[p, e, pl.ds(0, 16)]
                v = jnp.maximum(v, 0.2 * v)
                sb[p, e, pl.ds(0, 16)] = jnp.exp(v)
                return 0

            lax.fori_loop(0, KG, ebody, 0)
            base = wid * EW + g * KG
            pltpu.async_copy(sb.at[p], out_h.at[pl.ds(base, KG)], ws)

        fire(0, 0)

        def pair(i, _):
            for p in range(2):
                g = 2 * i + p

                @pl.when(g + 1 < NG)
                def _(g=g, p=p):
                    fire(g + 1, p ^ 1)

                @pl.when(g < NG)
                def _(g=g, p=p):
                    process(g, p)
            return 0

        lax.fori_loop(0, (NG + 1) // 2, pair, 0)
        for p in range(2):
            pltpu.make_async_copy(
                sb.at[p], out_h.at[pl.ds(wid * EW, KG)], ws).wait()

    return k(elt, ert, src3, dst3)


def _expand_body(w_ref, m_ref, wall_ref):
    wm = jnp.dot(w_ref[...], m_ref[...], preferred_element_type=jnp.float32)
    blk = wm.shape[0]
    wall_ref[...] = wm.reshape(blk, -1, 128).transpose(1, 0, 2)


def _expand(w16, fh, nb):
    m = _wall_mat(H, fh, nb)
    blk = 2000
    grid = E // blk
    return pl.pallas_call(
        _expand_body,
        grid=(grid,),
        in_specs=[
            pl.BlockSpec((blk, 16), lambda i: (i, 0)),
            pl.BlockSpec((16, nb * 128), lambda i: (0, 0)),
        ],
        out_specs=pl.BlockSpec((nb, blk, 128), lambda i: (0, i, 0)),
        out_shape=jax.ShapeDtypeStruct((nb, E, 128), jnp.float32),
    )(w16, m)


def _sc_agg(tb, wall, w16, src, dst, nb):
    mesh = plsc.VectorSubcoreMesh(core_axis_name="c", subcore_axis_name="s")
    rows_per = NPAD // 16
    zrows = jnp.zeros((NPAD, 128), jnp.float32)

    @functools.partial(
        pl.kernel,
        mesh=mesh,
        out_type=jax.ShapeDtypeStruct((nb + 1, 2 * NPAD, 128), jnp.float32),
        scratch_types=[
            pltpu.VMEM_SHARED((NPAD, 128), jnp.float32),
            pltpu.VMEM((NGC, KGA), jnp.int32),
            pltpu.VMEM((NGC, KGA), jnp.int32),
            pltpu.VMEM((2, KGA, 128), jnp.float32),
            pltpu.VMEM((2, KGA, 128), jnp.float32),
            pltpu.VMEM((2, KGA, 16), jnp.float32),
            pltpu.SemaphoreType.DMA,
            pltpu.SemaphoreType.DMA,
            pltpu.SemaphoreType.DMA,
        ],
    )
    def k(tb_h, wall_h, w16_h, src_h, dst_h, z_h, out_h, acc, sidx, didx,
          rows, wbuf, w16c, gs0, gs1, ss):
        cid = lax.axis_index("c")
        sid = lax.axis_index("s")
        wid = sid * 2 + cid
        myrow = sid * rows_per
        gsem = [gs0, gs1]
        for b in range(nb + 1):
            pltpu.sync_copy(z_h.at[pl.ds(myrow, rows_per)],
                            acc.at[pl.ds(myrow, rows_per)])
            plsc.subcore_barrier()

            gath = b < nb
            if not gath:
                def zbody(e, _):
                    for p in range(2):
                        for q in range(1, 8):
                            rows[p, e, pl.ds(q * 16, 16)] = jnp.zeros(
                                (16,), jnp.float32)
                    return 0

                lax.fori_loop(0, KGA, zbody, 0)

            def fire(c, j, p, b=b, gath=gath):
                base = wid * EW + (c * NGC + j) * KGA
                if gath:
                    pltpu.async_copy(tb_h.at[b].at[sidx.at[j]], rows.at[p],
                                     gsem[p])
                    pltpu.async_copy(wall_h.at[b].at[pl.ds(base, KGA)],
                                     wbuf.at[p], gsem[p])
                else:
                    pltpu.async_copy(w16_h.at[pl.ds(base, KGA)],
                                     w16c.at[p], gsem[p])

            def process(c, j, p, b=b, gath=gath):
                base = wid * EW + (c * NGC + j) * KGA
                if gath:
                    pltpu.make_async_copy(
                        tb_h.at[b].at[sidx.at[j]], rows.at[p], gsem[p]).wait()
                    pltpu.make_async_copy(
                        wall_h.at[b].at[pl.ds(base, KGA)], wbuf.at[p],
                        gsem[p]).wait()

                    def ebody(e, _):
                        for q in range(8):
                            sl = pl.ds(q * 16, 16)
                            rows[p, e, sl] = rows[p, e, sl] * wbuf[p, e, sl]
                        return 0

                    lax.fori_loop(0, KGA, ebody, 0)
                else:
                    pltpu.make_async_copy(
                        w16_h.at[pl.ds(base, KGA)], w16c.at[p], gsem[p]).wait()

                    def ebody(e, _):
                        rows[p, e, pl.ds(0, 16)] = w16c[p, e, pl.ds(0, 16)]
                        return 0

                    lax.fori_loop(0, KGA, ebody, 0)
                pltpu.async_copy(rows.at[p], acc.at[didx.at[j]], ss,
                                 add=True)

            def chunk(c, _):
                pltpu.sync_copy(src_h.at[wid].at[c], sidx)
                pltpu.sync_copy(dst_h.at[wid].at[c], didx)
                fire(c, 0, 0)

                def pair(i, _):
                    for p in range(2):
                        j = 2 * i + p

                        @pl.when(j + 1 < NGC)
                        def _(j=j, p=p):
                            @pl.when(j >= 1)
                            def _():
                                pltpu.make_async_copy(
                                    rows.at[p ^ 1], acc.at[didx.at[0]],
                                    ss).wait()

                            fire(c, j + 1, p ^ 1)

                        @pl.when(j < NGC)
                        def _(j=j, p=p):
                            process(c, j, p)
                    return 0

                lax.fori_loop(0, (NGC + 1) // 2, pair, 0)
                for p in range(2):
                    pltpu.make_async_copy(
                        rows.at[p], acc.at[didx.at[0]], ss).wait()
                return 0

            lax.fori_loop(0, NGA // NGC, chunk, 0)

            plsc.subcore_barrier()
            pltpu.sync_copy(
                acc.at[pl.ds(myrow, rows_per)],
                out_h.at[b].at[pl.ds(cid * NPAD + myrow, rows_per)])

    out = k(tb, wall, w16, src, dst, zrows)
    return out.reshape(nb + 1, 2, NPAD, 128)


def _dense2_body(a00, a01, a10, a11, d0, d1, rep_ref, b_ref, w2_ref, alm_ref,
                 arm_ref, tb_ref, elt_ref, ert_ref):
    numer = jnp.concatenate(
        [a00[...] + a01[...], a10[...] + a11[...]], axis=1)
    den8 = (d0[...] + d1[...])[:, :H]
    rep = jnp.dot(den8, rep_ref[...], preferred_element_type=jnp.float32)
    rep = jnp.where(rep == 0.0, 1.0, rep)
    o1 = numer / rep + b_ref[...]
    o1 = jnp.where(o1 > 0, o1, (jnp.exp(o1) - 1.0))
    h2 = jnp.dot(o1, w2_ref[...], preferred_element_type=jnp.float32)
    el = jnp.dot(h2, alm_ref[...], preferred_element_type=jnp.float32)
    er = jnp.dot(h2, arm_ref[...], preferred_element_type=jnp.float32)
    blk = h2.shape[0]
    tb_ref[...] = h2.reshape(blk, -1, 128).transpose(1, 0, 2)
    z = jnp.zeros((blk, 120), jnp.float32)
    elt_ref[...] = jnp.concatenate([el, z], axis=1)
    ert_ref[...] = jnp.concatenate([er, z], axis=1)


def _dense2(agg1, b1, W2, al2, ar2):
    alm = _expand_att(al2)
    arm = _expand_att(ar2)
    rep = _rep_mat(H, DH)
    nb2 = (H * DO) // 128
    blk = 1280
    grid = NPAD // blk
    row = pl.BlockSpec((blk, 128), lambda i: (i, 0))
    return pl.pallas_call(
        _dense2_body,
        grid=(grid,),
        in_specs=[
            row, row, row, row, row, row,
            pl.BlockSpec((H, H * DH), lambda i: (0, 0)),
            pl.BlockSpec((1, H * DH), lambda i: (0, 0)),
            pl.BlockSpec((H * DH, H * DO), lambda i: (0, 0)),
            pl.BlockSpec((H * DO, H), lambda i: (0, 0)),
            pl.BlockSpec((H * DO, H), lambda i: (0, 0)),
        ],
        out_specs=[
            pl.BlockSpec((nb2, blk, 128), lambda i: (0, i, 0)),
            pl.BlockSpec((blk, 128), lambda i: (i, 0)),
            pl.BlockSpec((blk, 128), lambda i: (i, 0)),
        ],
        out_shape=[
            jax.ShapeDtypeStruct((nb2, NPAD, 128), jnp.float32),
            jax.ShapeDtypeStruct((NPAD, 128), jnp.float32),
            jax.ShapeDtypeStruct((NPAD, 128), jnp.float32),
        ],
    )(agg1[0, 0], agg1[0, 1], agg1[1, 0], agg1[1, 1], agg1[2, 0], agg1[2, 1],
      rep, b1.reshape(1, -1), W2, alm, arm)


def _head_body(a00, a01, a10, a11, a20, a21, a30, a31, d0, d1, rep_ref, b_ref,
               w1_ref, b1_ref, w2_ref, b2_ref, w3_ref, b3_ref, out_ref):
    numer = jnp.concatenate(
        [a00[...] + a01[...], a10[...] + a11[...], a20[...] + a21[...],
         a30[...] + a31[...]], axis=1)
    den8 = (d0[...] + d1[...])[:, :H]
    rep = jnp.dot(den8, rep_ref[...], preferred_element_type=jnp.float32)
    rep = jnp.where(rep == 0.0, 1.0, rep)
    o2 = numer / rep + b_ref[...]
    o2 = jnp.where(o2 > 0, o2, (jnp.exp(o2) - 1.0))
    hh = jnp.dot(o2, w1_ref[...], preferred_element_type=jnp.float32) + b1_ref[...]
    hh = jnp.where(hh > 0, hh, 0.01 * hh)
    hh = jnp.dot(hh, w2_ref[...], preferred_element_type=jnp.float32) + b2_ref[...]
    hh = jnp.where(hh > 0, hh, 0.01 * hh)
    out_ref[...] = jnp.dot(hh, w3_ref[...], preferred_element_type=jnp.float32) + b3_ref[...]


def _head(agg2, b2, d1w, d1b, d2w, d2b, d3w, d3b):
    rep = _rep_mat(H, DO)
    blk = 400
    grid = N // blk
    row = pl.BlockSpec((blk, 128), lambda i: (i, 0))
    return pl.pallas_call(
        _head_body,
        grid=(grid,),
        in_specs=[
            row, row, row, row, row, row, row, row, row, row,
            pl.BlockSpec((H, H * DO), lambda i: (0, 0)),
            pl.BlockSpec((1, H * DO), lambda i: (0, 0)),
            pl.BlockSpec((H * DO, DO), lambda i: (0, 0)),
            pl.BlockSpec((1, DO), lambda i: (0, 0)),
            pl.BlockSpec((DO, DO // 2), lambda i: (0, 0)),
            pl.BlockSpec((1, DO // 2), lambda i: (0, 0)),
            pl.BlockSpec((DO // 2, 1), lambda i: (0, 0)),
            pl.BlockSpec((1, 1), lambda i: (0, 0)),
        ],
        out_specs=pl.BlockSpec((blk, 1), lambda i: (i, 0)),
        out_shape=jax.ShapeDtypeStruct((N, 1), jnp.float32),
    )(agg2[0, 0][:N], agg2[0, 1][:N], agg2[1, 0][:N], agg2[1, 1][:N],
      agg2[2, 0][:N], agg2[2, 1][:N], agg2[3, 0][:N], agg2[3, 1][:N],
      agg2[4, 0][:N], agg2[4, 1][:N], rep, b2.reshape(1, -1), d1w,
      d1b.reshape(1, -1), d2w, d2b.reshape(1, -1), d3w, d3b.reshape(1, -1))


def kernel(x, edge_index, W1, al1, ar1, b1, s1w1, s1b1, s1w2, W2, al2, ar2, b2,
           s2w1, s2b1, s2w2, d1w, d1b, d2w, d2b, d3w, d3b):
    src3 = edge_index[0].reshape(NW, NG, KG)
    dst3 = edge_index[1].reshape(NW, NG, KG)
    src4 = edge_index[0].reshape(NW, NGA // NGC, NGC, KGA)
    dst4 = edge_index[1].reshape(NW, NGA // NGC, NGC, KGA)
    nb1 = (H * DH) // 128
    nb2 = (H * DO) // 128

    tb1, elt1, ert1 = _dense1(x, W1, al1, ar1)
    w16_1 = _sc_logits(elt1, ert1, src3, dst3)
    wall1 = _expand(w16_1, DH, nb1)
    agg1 = _sc_agg(tb1, wall1, w16_1, src4, dst4, nb1)

    tb2, elt2, ert2 = _dense2(agg1, b1, W2, al2, ar2)
    w16_2 = _sc_logits(elt2, ert2, src3, dst3)
    wall2 = _expand(w16_2, DO, nb2)
    agg2 = _sc_agg(tb2, wall2, w16_2, src4, dst4, nb2)

    return _head(agg2, b2, d1w, d1b, d2w, d2b, d3w, d3b)
